# baseline (speedup 1.0000x reference)
import functools

import jax
import jax.numpy as jnp
from jax import lax
from jax.experimental import pallas as pl
from jax.experimental.pallas import tpu as pltpu
from jax.experimental.pallas import tpu_sc as plsc

NC = 2
NS = 16
NW = NC * NS
CHUNK = 128


def _silu(v):
    return v * jax.nn.sigmoid(v)


def _sc_mesh():
    return plsc.VectorSubcoreMesh(core_axis_name="c", subcore_axis_name="s")


def _sc_radial(x_flat, row_p, col_p):
    e_pad = row_p.shape[0]
    cpw = e_pad // CHUNK // NW

    def body(x_hbm, row_hbm, col_hbm, out_hbm, x_v, ri_v, ci_v, rad_v):
        w = lax.axis_index("s") * NC + lax.axis_index("c")
        pltpu.sync_copy(x_hbm, x_v)

        def chunk(cidx, carry):
            base = (w * cpw + cidx) * CHUNK
            pltpu.sync_copy(row_hbm.at[pl.ds(base, CHUNK)], ri_v)
            pltpu.sync_copy(col_hbm.at[pl.ds(base, CHUNK)], ci_v)
            for j in range(CHUNK // 16):
                r16 = ri_v[pl.ds(j * 16, 16)] * 3
                c16 = ci_v[pl.ds(j * 16, 16)] * 3
                acc = jnp.zeros((16,), jnp.float32)
                for comp in range(3):
                    a = plsc.load_gather(x_v, [r16 + comp])
                    b = plsc.load_gather(x_v, [c16 + comp])
                    d = a - b
                    acc = acc + d * d
                rad_v[pl.ds(j * 16, 16)] = acc
            pltpu.sync_copy(rad_v, out_hbm.at[pl.ds(base, CHUNK)])
            return carry

        lax.fori_loop(0, cpw, chunk, 0)

    return pl.kernel(
        body,
        out_type=jax.ShapeDtypeStruct((e_pad,), jnp.float32),
        mesh=_sc_mesh(),
        compiler_params=pltpu.CompilerParams(needs_layout_passes=False),
        scratch_types=[
            pltpu.VMEM((x_flat.shape[0],), jnp.float32),
            pltpu.VMEM((CHUNK,), jnp.int32),
            pltpu.VMEM((CHUNK,), jnp.int32),
            pltpu.VMEM((CHUNK,), jnp.float32),
        ],
    )(x_flat, row_p, col_p)


def _sc_gather(pa, pb, row_2d, col_2d):
    n, h = pa.shape
    nrow2d = row_2d.shape[0]
    e_pad = nrow2d * CHUNK
    C = 64
    nch = e_pad // C // NW
    irows = nch * C // CHUNK

    def body(pa_hbm, pb_hbm, row_hbm, col_hbm, ga_hbm, gb_hbm,
             ri_v, ci_v, a0, a1, b0, b1,
             sa0, sa1, sb0, sb1, soa0, soa1, sob0, sob1):
        w = lax.axis_index("s") * NC + lax.axis_index("c")
        a_ = [a0, a1]
        b_ = [b0, b1]
        sa_ = [sa0, sa1]
        sb_ = [sb0, sb1]
        soa_ = [soa0, soa1]
        sob_ = [sob0, sob1]

        pltpu.sync_copy(row_hbm.at[pl.ds(w * irows, irows)], ri_v)
        pltpu.sync_copy(col_hbm.at[pl.ds(w * irows, irows)], ci_v)

        def idx_ref(iv, cidx):
            r = cidx // 2
            half = cidx % 2
            return iv.at[r, pl.ds(half * C, C)]

        def issue(cidx, slot):
            pltpu.async_copy(pa_hbm.at[idx_ref(ri_v, cidx)], a_[slot], sa_[slot])
            pltpu.async_copy(pb_hbm.at[idx_ref(ci_v, cidx)], b_[slot], sb_[slot])

        issue(0, 0)

        def step(c, slot):
            nxt = 1 - slot

            @pl.when(c + 1 < nch)
            def _():
                @pl.when(c >= 1)
                def _():
                    pltpu.make_async_copy(a_[nxt], ga_hbm.at[pl.ds(0, C)],
                                          soa_[nxt]).wait()
                    pltpu.make_async_copy(b_[nxt], gb_hbm.at[pl.ds(0, C)],
                                          sob_[nxt]).wait()

                issue(c + 1, nxt)

            pltpu.make_async_copy(pa_hbm.at[idx_ref(ri_v, c)], a_[slot],
                                  sa_[slot]).wait()
            pltpu.make_async_copy(pb_hbm.at[idx_ref(ci_v, c)], b_[slot],
                                  sb_[slot]).wait()
            base = w * nch * C + c * C
            pltpu.async_copy(a_[slot], ga_hbm.at[pl.ds(base, C)], soa_[slot])
            pltpu.async_copy(b_[slot], gb_hbm.at[pl.ds(base, C)], sob_[slot])

        def pair(i, carry):
            step(2 * i, 0)
            step(2 * i + 1, 1)
            return carry

        lax.fori_loop(0, nch // 2, pair, 0)
        pltpu.make_async_copy(a0, ga_hbm.at[pl.ds(0, C)], soa0).wait()
        pltpu.make_async_copy(b0, gb_hbm.at[pl.ds(0, C)], sob0).wait()
        pltpu.make_async_copy(a1, ga_hbm.at[pl.ds(0, C)], soa1).wait()
        pltpu.make_async_copy(b1, gb_hbm.at[pl.ds(0, C)], sob1).wait()

    dma = pltpu.SemaphoreType.DMA
    return pl.kernel(
        body,
        out_type=(jax.ShapeDtypeStruct((e_pad, h), jnp.float32),
                  jax.ShapeDtypeStruct((e_pad, h), jnp.float32)),
        mesh=_sc_mesh(),
        scratch_types=[
            pltpu.VMEM((irows, CHUNK), jnp.int32),
            pltpu.VMEM((irows, CHUNK), jnp.int32),
            pltpu.VMEM((C, h), jnp.float32),
            pltpu.VMEM((C, h), jnp.float32),
            pltpu.VMEM((C, h), jnp.float32),
            pltpu.VMEM((C, h), jnp.float32),
            dma, dma, dma, dma, dma, dma, dma, dma,
        ],
    )(pa, pb, row_2d, col_2d)


def _sc_scatter(m2, row_2d, n):
    e_pad, h = m2.shape
    hc = h // NC
    cpt = e_pad // CHUNK // NS
    br = 80
    nblk = n // br
    bpt = (nblk + NS - 1) // NS

    def body(m2_hbm, row_hbm, agg_hbm, idx_v, m0, m1, acc_sh,
             sr0, sr1, sw0, sw1):
        c = lax.axis_index("c")
        s = lax.axis_index("s")
        colbase = c * hc
        m_ = [m0, m1]
        sr_ = [sr0, sr1]
        sw_ = [sw0, sw1]
        zero16 = jnp.zeros((16,), jnp.float32)

        pltpu.sync_copy(row_hbm.at[pl.ds(s * cpt, cpt)], idx_v)

        def zrow(r, c2):
            for k in range(hc // 16):
                m0[r, pl.ds(k * 16, 16)] = zero16
            return c2

        lax.fori_loop(0, br, zrow, 0)

        def zcp(i, c2):
            blk = s + NS * i

            @pl.when(blk < nblk)
            def _():
                pltpu.sync_copy(m0.at[pl.ds(0, br)],
                                acc_sh.at[pl.ds(blk * br, br)])

            return c2

        lax.fori_loop(0, bpt, zcp, 0)
        plsc.subcore_barrier()

        def issue(cidx, slot):
            base = (s * cpt + cidx) * CHUNK
            pltpu.async_copy(
                m2_hbm.at[pl.ds(base, CHUNK), pl.ds(colbase, hc)],
                m_[slot], sr_[slot])

        issue(0, 0)

        def step(cidx, slot):
            nxt = 1 - slot

            @pl.when(cidx >= 1)
            def _():
                pltpu.make_async_copy(m_[nxt], acc_sh.at[idx_v.at[cidx]],
                                      sw_[nxt]).wait()

            @pl.when(cidx + 1 < cpt)
            def _():
                issue(cidx + 1, nxt)

            base = (s * cpt + cidx) * CHUNK
            pltpu.make_async_copy(
                m2_hbm.at[pl.ds(base, CHUNK), pl.ds(colbase, hc)],
                m_[slot], sr_[slot]).wait()

            pltpu.async_copy(m_[slot], acc_sh.at[idx_v.at[cidx]],
                             sw_[slot], add=True)

        def pair(i, carry):
            step(2 * i, 0)
            step(2 * i + 1, 1)
            return carry

        lax.fori_loop(0, cpt // 2, pair, 0)
        pltpu.make_async_copy(m1, acc_sh.at[idx_v.at[0]], sw1).wait()
        plsc.subcore_barrier()

        def wb(i, c2):
            blk = s + NS * i

            @pl.when(blk < nblk)
            def _():
                rb = blk * br
                pltpu.sync_copy(acc_sh.at[pl.ds(rb, br)], m0.at[pl.ds(0, br)])
                pltpu.sync_copy(m0.at[pl.ds(0, br)],
                                agg_hbm.at[pl.ds(rb, br), pl.ds(colbase, hc)])

            return c2

        lax.fori_loop(0, bpt, wb, 0)

    dma = pltpu.SemaphoreType.DMA
    return pl.kernel(
        body,
        out_type=jax.ShapeDtypeStruct((n, h), jnp.float32),
        mesh=_sc_mesh(),
        scratch_types=[
            pltpu.VMEM((cpt, CHUNK), jnp.int32),
            pltpu.VMEM((CHUNK, hc), jnp.float32),
            pltpu.VMEM((CHUNK, hc), jnp.float32),
            pltpu.VMEM_SHARED((n, hc), jnp.float32),
            dma, dma, dma, dma,
        ],
    )(m2, row_2d)


def _emb(h0, w, b, bn=400):
    n, k = h0.shape
    h = w.shape[1]

    def body(x_ref, w_ref, b_ref, o_ref):
        o_ref[...] = (
            jnp.dot(x_ref[...], w_ref[...], preferred_element_type=jnp.float32, precision=lax.Precision.HIGHEST)
            + b_ref[...]
        )

    return pl.pallas_call(
        body,
        grid=(n // bn,),
        in_specs=[
            pl.BlockSpec((bn, k), lambda i: (i, 0)),
            pl.BlockSpec((k, h), lambda i: (0, 0)),
            pl.BlockSpec((1, h), lambda i: (0, 0)),
        ],
        out_specs=pl.BlockSpec((bn, h), lambda i: (i, 0)),
        out_shape=jax.ShapeDtypeStruct((n, h), jnp.float32),
    )(h0, w, b.reshape(1, -1))


def _pab(h, wa, wb, bn=400):
    n, hd = h.shape

    def body(h_ref, wa_ref, wb_ref, pa_ref, pb_ref):
        hv = h_ref[...]
        pa_ref[...] = jnp.dot(hv, wa_ref[...], preferred_element_type=jnp.float32, precision=lax.Precision.HIGHEST)
        pb_ref[...] = jnp.dot(hv, wb_ref[...], preferred_element_type=jnp.float32, precision=lax.Precision.HIGHEST)

    return pl.pallas_call(
        body,
        grid=(n // bn,),
        in_specs=[
            pl.BlockSpec((bn, hd), lambda i: (i, 0)),
            pl.BlockSpec((hd, hd), lambda i: (0, 0)),
            pl.BlockSpec((hd, hd), lambda i: (0, 0)),
        ],
        out_specs=[
            pl.BlockSpec((bn, hd), lambda i: (i, 0)),
            pl.BlockSpec((bn, hd), lambda i: (i, 0)),
        ],
        out_shape=[
            jax.ShapeDtypeStruct((n, hd), jnp.float32),
            jax.ShapeDtypeStruct((n, hd), jnp.float32),
        ],
    )(h, wa, wb)


def _edge_mlp(ga, gb, ea_aug, waug, b1, w2, b2, em_p, be=640):
    e_pad, hd = ga.shape
    ka = ea_aug.shape[1]

    def body(ga_ref, gb_ref, ea_ref, waug_ref, b1_ref, w2_ref, b2_ref, em_ref,
             o_ref):
        m1 = (
            ga_ref[...] + gb_ref[...]
            + jnp.dot(ea_ref[...], waug_ref[...], preferred_element_type=jnp.float32, precision=lax.Precision.HIGHEST)
            + b1_ref[...]
        )
        m1 = _silu(m1)
        m2 = _silu(
            jnp.dot(m1, w2_ref[...], preferred_element_type=jnp.float32, precision=lax.Precision.HIGHEST) + b2_ref[...]
        )
        o_ref[...] = m2 * em_ref[...]

    return pl.pallas_call(
        body,
        grid=(e_pad // be,),
        in_specs=[
            pl.BlockSpec((be, hd), lambda i: (i, 0)),
            pl.BlockSpec((be, hd), lambda i: (i, 0)),
            pl.BlockSpec((be, ka), lambda i: (i, 0)),
            pl.BlockSpec((ka, hd), lambda i: (0, 0)),
            pl.BlockSpec((1, hd), lambda i: (0, 0)),
            pl.BlockSpec((hd, hd), lambda i: (0, 0)),
            pl.BlockSpec((1, hd), lambda i: (0, 0)),
            pl.BlockSpec((be, 1), lambda i: (i, 0)),
        ],
        out_specs=pl.BlockSpec((be, hd), lambda i: (i, 0)),
        out_shape=jax.ShapeDtypeStruct((e_pad, hd), jnp.float32),
    )(ga, gb, ea_aug, waug, b1.reshape(1, -1), w2, b2.reshape(1, -1), em_p)


def _node_mlp(h, agg, h0, w1h, w1a, w1n, b1, w2, b2, bn=400):
    n, hd = h.shape
    kn = h0.shape[1]

    def body(h_ref, a_ref, n_ref, w1h_ref, w1a_ref, w1n_ref, b1_ref, w2_ref,
             b2_ref, o_ref):
        t = (
            jnp.dot(h_ref[...], w1h_ref[...], preferred_element_type=jnp.float32, precision=lax.Precision.HIGHEST)
            + jnp.dot(a_ref[...], w1a_ref[...], preferred_element_type=jnp.float32, precision=lax.Precision.HIGHEST)
            + jnp.dot(n_ref[...], w1n_ref[...], preferred_element_type=jnp.float32, precision=lax.Precision.HIGHEST)
            + b1_ref[...]
        )
        t = _silu(t)
        o_ref[...] = (
            h_ref[...]
            + jnp.dot(t, w2_ref[...], preferred_element_type=jnp.float32, precision=lax.Precision.HIGHEST)
            + b2_ref[...]
        )

    return pl.pallas_call(
        body,
        grid=(n // bn,),
        in_specs=[
            pl.BlockSpec((bn, hd), lambda i: (i, 0)),
            pl.BlockSpec((bn, hd), lambda i: (i, 0)),
            pl.BlockSpec((bn, kn), lambda i: (i, 0)),
            pl.BlockSpec((hd, hd), lambda i: (0, 0)),
            pl.BlockSpec((hd, hd), lambda i: (0, 0)),
            pl.BlockSpec((kn, hd), lambda i: (0, 0)),
            pl.BlockSpec((1, hd), lambda i: (0, 0)),
            pl.BlockSpec((hd, hd), lambda i: (0, 0)),
            pl.BlockSpec((1, hd), lambda i: (0, 0)),
        ],
        out_specs=pl.BlockSpec((bn, hd), lambda i: (i, 0)),
        out_shape=jax.ShapeDtypeStruct((n, hd), jnp.float32),
    )(h, agg, h0, w1h, w1a, w1n, b1.reshape(1, -1), w2, b2.reshape(1, -1))


def _dec(h, w1, b1, w2, b2, node_mask, nn, bn=400):
    n, hd = h.shape

    def body(h_ref, w1_ref, b1_ref, w2_ref, b2_ref, nm_ref, nn_ref, o_ref):
        t = _silu(
            jnp.dot(h_ref[...], w1_ref[...], preferred_element_type=jnp.float32, precision=lax.Precision.HIGHEST)
            + b1_ref[...]
        )
        o = jnp.dot(t, w2_ref[...], preferred_element_type=jnp.float32, precision=lax.Precision.HIGHEST) + b2_ref[...]
        base = pl.program_id(0) * bn
        rows = base + lax.broadcasted_iota(jnp.int32, (bn, 1), 0)
        valid = (rows < nn_ref[0, 0]).astype(jnp.float32)
        o_ref[...] = o * nm_ref[...] * valid

    return pl.pallas_call(
        body,
        grid=(n // bn,),
        in_specs=[
            pl.BlockSpec((bn, hd), lambda i: (i, 0)),
            pl.BlockSpec((hd, hd), lambda i: (0, 0)),
            pl.BlockSpec((1, hd), lambda i: (0, 0)),
            pl.BlockSpec((hd, hd), lambda i: (0, 0)),
            pl.BlockSpec((1, hd), lambda i: (0, 0)),
            pl.BlockSpec((bn, 1), lambda i: (i, 0)),
            pl.BlockSpec((1, 1), lambda i: (0, 0)),
        ],
        out_specs=pl.BlockSpec((bn, hd), lambda i: (i, 0)),
        out_shape=jax.ShapeDtypeStruct((n, hd), jnp.float32),
    )(h, w1, b1.reshape(1, -1), w2, b2.reshape(1, -1), node_mask,
      jnp.asarray(nn, jnp.int32).reshape(1, 1))


def kernel(h0, x, edge_attr, node_mask, edge_mask, emb_w, emb_b, edge_w1,
           edge_b1, edge_w2, edge_b2, node_w1, node_b1, node_w2, node_b2,
           dec_w1, dec_b1, dec_w2, dec_b2, edges, n_nodes):
    n, _ = h0.shape
    e = edges.shape[1]
    hd = emb_w.shape[1]
    lnum = edge_w1.shape[0]

    step = CHUNK * NW
    e_pad = ((e + step - 1) // step) * step
    pad = e_pad - e
    row_p = jnp.concatenate([edges[0], jnp.zeros((pad,), edges.dtype)])
    col_p = jnp.concatenate([edges[1], jnp.zeros((pad,), edges.dtype)])
    em_p = jnp.concatenate(
        [edge_mask, jnp.zeros((pad, 1), edge_mask.dtype)], axis=0)
    ea_p = jnp.concatenate(
        [edge_attr, jnp.zeros((pad, edge_attr.shape[1]), edge_attr.dtype)], axis=0)

    radial = _sc_radial(x.reshape(-1), row_p, col_p)
    ea_aug = jnp.concatenate([radial[:, None], ea_p], axis=1)
    row_2d = row_p.reshape(-1, CHUNK)
    col_2d = col_p.reshape(-1, CHUNK)

    h = _emb(h0, emb_w, emb_b)
    for i in range(lnum):
        wa = edge_w1[i, :hd]
        wb = edge_w1[i, hd:2 * hd]
        waug = edge_w1[i, 2 * hd:]
        pa, pb = _pab(h, wa, wb)
        ga, gb = _sc_gather(pa, pb, row_2d, col_2d)
        m2 = _edge_mlp(ga, gb, ea_aug, waug, edge_b1[i], edge_w2[i],
                       edge_b2[i], em_p)
        agg = _sc_scatter(m2, row_2d, n)
        h = _node_mlp(h, agg, h0, node_w1[i, :hd], node_w1[i, hd:2 * hd],
                      node_w1[i, 2 * hd:], node_b1[i], node_w2[i], node_b2[i])

    return _dec(h, dec_w1, dec_b1, dec_w2, dec_b2, node_mask, n_nodes)


# speedup vs baseline: 1.5465x; 1.5465x over previous
import functools

import jax
import jax.numpy as jnp
from jax import lax
from jax.experimental import pallas as pl
from jax.experimental.pallas import tpu as pltpu
from jax.experimental.pallas import tpu_sc as plsc

NC = 2
NS = 16
NW = NC * NS
CHUNK = 128


def _silu(v):
    return v * jax.nn.sigmoid(v)


def _sc_mesh():
    return plsc.VectorSubcoreMesh(core_axis_name="c", subcore_axis_name="s")


def _sc_radial(x_flat, row_p, col_p):
    e_pad = row_p.shape[0]
    cpw = e_pad // CHUNK // NW

    def body(x_hbm, row_hbm, col_hbm, out_hbm, x_v, ri_v, ci_v, rad_v):
        w = lax.axis_index("s") * NC + lax.axis_index("c")
        pltpu.sync_copy(x_hbm, x_v)

        def chunk(cidx, carry):
            base = (w * cpw + cidx) * CHUNK
            pltpu.sync_copy(row_hbm.at[pl.ds(base, CHUNK)], ri_v)
            pltpu.sync_copy(col_hbm.at[pl.ds(base, CHUNK)], ci_v)
            for j in range(CHUNK // 16):
                r16 = ri_v[pl.ds(j * 16, 16)] * 3
                c16 = ci_v[pl.ds(j * 16, 16)] * 3
                acc = jnp.zeros((16,), jnp.float32)
                for comp in range(3):
                    a = plsc.load_gather(x_v, [r16 + comp])
                    b = plsc.load_gather(x_v, [c16 + comp])
                    d = a - b
                    acc = acc + d * d
                rad_v[pl.ds(j * 16, 16)] = acc
            pltpu.sync_copy(rad_v, out_hbm.at[pl.ds(base, CHUNK)])
            return carry

        lax.fori_loop(0, cpw, chunk, 0)

    return pl.kernel(
        body,
        out_type=jax.ShapeDtypeStruct((e_pad,), jnp.float32),
        mesh=_sc_mesh(),
        compiler_params=pltpu.CompilerParams(needs_layout_passes=False),
        scratch_types=[
            pltpu.VMEM((x_flat.shape[0],), jnp.float32),
            pltpu.VMEM((CHUNK,), jnp.int32),
            pltpu.VMEM((CHUNK,), jnp.int32),
            pltpu.VMEM((CHUNK,), jnp.float32),
        ],
    )(x_flat, row_p, col_p)


def _sc_gather(pa, pb, row_2d, col_2d):
    n, h = pa.shape
    nrow2d = row_2d.shape[0]
    e_pad = nrow2d * CHUNK
    C = 32
    NB = 4
    K = 3
    nch = e_pad // C // NW
    cpr = CHUNK // C
    irows = nch // cpr

    def body(pa_hbm, pb_hbm, row_hbm, col_hbm, g_hbm,
             ri_v, ci_v, a0, a1, a2, a3, b0, b1, b2, b3, o0, o1, o2, o3,
             sa0, sa1, sa2, sa3, sb0, sb1, sb2, sb3, so0, so1, so2, so3):
        w = lax.axis_index("s") * NC + lax.axis_index("c")
        a_ = [a0, a1, a2, a3]
        b_ = [b0, b1, b2, b3]
        o_ = [o0, o1, o2, o3]
        sa_ = [sa0, sa1, sa2, sa3]
        sb_ = [sb0, sb1, sb2, sb3]
        so_ = [so0, so1, so2, so3]

        pltpu.sync_copy(row_hbm.at[pl.ds(w * irows, irows)], ri_v)
        pltpu.sync_copy(col_hbm.at[pl.ds(w * irows, irows)], ci_v)

        def idx_ref(iv, cidx):
            r = cidx // cpr
            sub = cidx % cpr
            return iv.at[r, pl.ds(sub * C, C)]

        def issue(cidx, slot):
            pltpu.async_copy(pa_hbm.at[idx_ref(ri_v, cidx)], a_[slot], sa_[slot])
            pltpu.async_copy(pb_hbm.at[idx_ref(ci_v, cidx)], b_[slot], sb_[slot])

        for p in range(K):
            issue(p, p)

        def step(c, slot):
            tgt = c + K
            tslot = (slot + K) % NB

            @pl.when(tgt < nch)
            def _():
                issue(tgt, tslot)

            pltpu.make_async_copy(pa_hbm.at[idx_ref(ri_v, c)], a_[slot],
                                  sa_[slot]).wait()
            pltpu.make_async_copy(pb_hbm.at[idx_ref(ci_v, c)], b_[slot],
                                  sb_[slot]).wait()

            @pl.when(c >= NB)
            def _():
                pltpu.make_async_copy(o_[slot], g_hbm.at[pl.ds(0, C)],
                                      so_[slot]).wait()

            def addrow(r, c2):
                for k in range(h // 16):
                    sl = pl.ds(k * 16, 16)
                    o_[slot][r, sl] = a_[slot][r, sl] + b_[slot][r, sl]
                return c2

            lax.fori_loop(0, C, addrow, 0)
            base = w * nch * C + c * C
            pltpu.async_copy(o_[slot], g_hbm.at[pl.ds(base, C)], so_[slot])

        def quad(i, carry):
            for p in range(NB):
                step(NB * i + p, p)
            return carry

        lax.fori_loop(0, nch // NB, quad, 0)
        for p in range(NB):
            pltpu.make_async_copy(o_[p], g_hbm.at[pl.ds(0, C)], so_[p]).wait()

    dma = pltpu.SemaphoreType.DMA
    buf = pltpu.VMEM((C, h), jnp.float32)
    return pl.kernel(
        body,
        out_type=jax.ShapeDtypeStruct((e_pad, h), jnp.float32),
        mesh=_sc_mesh(),
        scratch_types=[
            pltpu.VMEM((irows, CHUNK), jnp.int32),
            pltpu.VMEM((irows, CHUNK), jnp.int32),
            buf, buf, buf, buf, buf, buf, buf, buf, buf, buf, buf, buf,
            dma, dma, dma, dma, dma, dma, dma, dma, dma, dma, dma, dma,
        ],
    )(pa, pb, row_2d, col_2d)


def _sc_scatter(m2, row_2d, n):
    e_pad, h = m2.shape
    hc = h // NC
    cpt = e_pad // CHUNK // NS
    br = 80
    nblk = n // br
    bpt = (nblk + NS - 1) // NS

    def body(m2_hbm, row_hbm, agg_hbm, idx_v, m0, m1, acc_sh,
             sr0, sr1, sw0, sw1):
        c = lax.axis_index("c")
        s = lax.axis_index("s")
        colbase = c * hc
        m_ = [m0, m1]
        sr_ = [sr0, sr1]
        sw_ = [sw0, sw1]
        zero16 = jnp.zeros((16,), jnp.float32)

        pltpu.sync_copy(row_hbm.at[pl.ds(s * cpt, cpt)], idx_v)

        def zrow(r, c2):
            for k in range(hc // 16):
                m0[r, pl.ds(k * 16, 16)] = zero16
            return c2

        lax.fori_loop(0, br, zrow, 0)

        def zcp(i, c2):
            blk = s + NS * i

            @pl.when(blk < nblk)
            def _():
                pltpu.sync_copy(m0.at[pl.ds(0, br)],
                                acc_sh.at[pl.ds(blk * br, br)])

            return c2

        lax.fori_loop(0, bpt, zcp, 0)
        plsc.subcore_barrier()

        def issue(cidx, slot):
            base = (s * cpt + cidx) * CHUNK
            pltpu.async_copy(
                m2_hbm.at[pl.ds(base, CHUNK), pl.ds(colbase, hc)],
                m_[slot], sr_[slot])

        issue(0, 0)

        def step(cidx, slot):
            nxt = 1 - slot

            @pl.when(cidx >= 1)
            def _():
                pltpu.make_async_copy(m_[nxt], acc_sh.at[idx_v.at[cidx]],
                                      sw_[nxt]).wait()

            @pl.when(cidx + 1 < cpt)
            def _():
                issue(cidx + 1, nxt)

            base = (s * cpt + cidx) * CHUNK
            pltpu.make_async_copy(
                m2_hbm.at[pl.ds(base, CHUNK), pl.ds(colbase, hc)],
                m_[slot], sr_[slot]).wait()

            pltpu.async_copy(m_[slot], acc_sh.at[idx_v.at[cidx]],
                             sw_[slot], add=True)

        def pair(i, carry):
            step(2 * i, 0)
            step(2 * i + 1, 1)
            return carry

        lax.fori_loop(0, cpt // 2, pair, 0)
        pltpu.make_async_copy(m1, acc_sh.at[idx_v.at[0]], sw1).wait()
        plsc.subcore_barrier()

        def wb(i, c2):
            blk = s + NS * i

            @pl.when(blk < nblk)
            def _():
                rb = blk * br
                pltpu.sync_copy(acc_sh.at[pl.ds(rb, br)], m0.at[pl.ds(0, br)])
                pltpu.sync_copy(m0.at[pl.ds(0, br)],
                                agg_hbm.at[pl.ds(rb, br), pl.ds(colbase, hc)])

            return c2

        lax.fori_loop(0, bpt, wb, 0)

    dma = pltpu.SemaphoreType.DMA
    return pl.kernel(
        body,
        out_type=jax.ShapeDtypeStruct((n, h), jnp.float32),
        mesh=_sc_mesh(),
        scratch_types=[
            pltpu.VMEM((cpt, CHUNK), jnp.int32),
            pltpu.VMEM((CHUNK, hc), jnp.float32),
            pltpu.VMEM((CHUNK, hc), jnp.float32),
            pltpu.VMEM_SHARED((n, hc), jnp.float32),
            dma, dma, dma, dma,
        ],
    )(m2, row_2d)


def _emb(h0, w, b, bn=400):
    n, k = h0.shape
    h = w.shape[1]

    def body(x_ref, w_ref, b_ref, o_ref):
        o_ref[...] = (
            jnp.dot(x_ref[...], w_ref[...], preferred_element_type=jnp.float32)
            + b_ref[...]
        )

    return pl.pallas_call(
        body,
        grid=(n // bn,),
        in_specs=[
            pl.BlockSpec((bn, k), lambda i: (i, 0)),
            pl.BlockSpec((k, h), lambda i: (0, 0)),
            pl.BlockSpec((1, h), lambda i: (0, 0)),
        ],
        out_specs=pl.BlockSpec((bn, h), lambda i: (i, 0)),
        out_shape=jax.ShapeDtypeStruct((n, h), jnp.float32),
    )(h0, w, b.reshape(1, -1))


def _pab(h, wa, wb, bn=400):
    n, hd = h.shape

    def body(h_ref, wa_ref, wb_ref, pa_ref, pb_ref):
        hv = h_ref[...]
        pa_ref[...] = jnp.dot(hv, wa_ref[...], preferred_element_type=jnp.float32)
        pb_ref[...] = jnp.dot(hv, wb_ref[...], preferred_element_type=jnp.float32)

    return pl.pallas_call(
        body,
        grid=(n // bn,),
        in_specs=[
            pl.BlockSpec((bn, hd), lambda i: (i, 0)),
            pl.BlockSpec((hd, hd), lambda i: (0, 0)),
            pl.BlockSpec((hd, hd), lambda i: (0, 0)),
        ],
        out_specs=[
            pl.BlockSpec((bn, hd), lambda i: (i, 0)),
            pl.BlockSpec((bn, hd), lambda i: (i, 0)),
        ],
        out_shape=[
            jax.ShapeDtypeStruct((n, hd), jnp.float32),
            jax.ShapeDtypeStruct((n, hd), jnp.float32),
        ],
    )(h, wa, wb)


def _edge_mlp(g, ea_aug, waug, b1, w2, b2, em_p, be=640):
    e_pad, hd = g.shape
    ka = ea_aug.shape[1]

    def body(g_ref, ea_ref, waug_ref, b1_ref, w2_ref, b2_ref, em_ref, o_ref):
        m1 = (
            g_ref[...]
            + jnp.dot(ea_ref[...], waug_ref[...], preferred_element_type=jnp.float32)
            + b1_ref[...]
        )
        m1 = _silu(m1)
        m2 = _silu(
            jnp.dot(m1, w2_ref[...], preferred_element_type=jnp.float32) + b2_ref[...]
        )
        o_ref[...] = m2 * em_ref[...]

    return pl.pallas_call(
        body,
        grid=(e_pad // be,),
        in_specs=[
            pl.BlockSpec((be, hd), lambda i: (i, 0)),
            pl.BlockSpec((be, ka), lambda i: (i, 0)),
            pl.BlockSpec((ka, hd), lambda i: (0, 0)),
            pl.BlockSpec((1, hd), lambda i: (0, 0)),
            pl.BlockSpec((hd, hd), lambda i: (0, 0)),
            pl.BlockSpec((1, hd), lambda i: (0, 0)),
            pl.BlockSpec((be, 1), lambda i: (i, 0)),
        ],
        out_specs=pl.BlockSpec((be, hd), lambda i: (i, 0)),
        out_shape=jax.ShapeDtypeStruct((e_pad, hd), jnp.float32),
    )(g, ea_aug, waug, b1.reshape(1, -1), w2, b2.reshape(1, -1), em_p)


def _node_mlp(h, agg, h0, w1h, w1a, w1n, b1, w2, b2, bn=400):
    n, hd = h.shape
    kn = h0.shape[1]

    def body(h_ref, a_ref, n_ref, w1h_ref, w1a_ref, w1n_ref, b1_ref, w2_ref,
             b2_ref, o_ref):
        t = (
            jnp.dot(h_ref[...], w1h_ref[...], preferred_element_type=jnp.float32)
            + jnp.dot(a_ref[...], w1a_ref[...], preferred_element_type=jnp.float32)
            + jnp.dot(n_ref[...], w1n_ref[...], preferred_element_type=jnp.float32)
            + b1_ref[...]
        )
        t = _silu(t)
        o_ref[...] = (
            h_ref[...]
            + jnp.dot(t, w2_ref[...], preferred_element_type=jnp.float32)
            + b2_ref[...]
        )

    return pl.pallas_call(
        body,
        grid=(n // bn,),
        in_specs=[
            pl.BlockSpec((bn, hd), lambda i: (i, 0)),
            pl.BlockSpec((bn, hd), lambda i: (i, 0)),
            pl.BlockSpec((bn, kn), lambda i: (i, 0)),
            pl.BlockSpec((hd, hd), lambda i: (0, 0)),
            pl.BlockSpec((hd, hd), lambda i: (0, 0)),
            pl.BlockSpec((kn, hd), lambda i: (0, 0)),
            pl.BlockSpec((1, hd), lambda i: (0, 0)),
            pl.BlockSpec((hd, hd), lambda i: (0, 0)),
            pl.BlockSpec((1, hd), lambda i: (0, 0)),
        ],
        out_specs=pl.BlockSpec((bn, hd), lambda i: (i, 0)),
        out_shape=jax.ShapeDtypeStruct((n, hd), jnp.float32),
    )(h, agg, h0, w1h, w1a, w1n, b1.reshape(1, -1), w2, b2.reshape(1, -1))


def _dec(h, w1, b1, w2, b2, node_mask, nn, bn=400):
    n, hd = h.shape

    def body(h_ref, w1_ref, b1_ref, w2_ref, b2_ref, nm_ref, nn_ref, o_ref):
        t = _silu(
            jnp.dot(h_ref[...], w1_ref[...], preferred_element_type=jnp.float32)
            + b1_ref[...]
        )
        o = jnp.dot(t, w2_ref[...], preferred_element_type=jnp.float32) + b2_ref[...]
        base = pl.program_id(0) * bn
        rows = base + lax.broadcasted_iota(jnp.int32, (bn, 1), 0)
        valid = (rows < nn_ref[0, 0]).astype(jnp.float32)
        o_ref[...] = o * nm_ref[...] * valid

    return pl.pallas_call(
        body,
        grid=(n // bn,),
        in_specs=[
            pl.BlockSpec((bn, hd), lambda i: (i, 0)),
            pl.BlockSpec((hd, hd), lambda i: (0, 0)),
            pl.BlockSpec((1, hd), lambda i: (0, 0)),
            pl.BlockSpec((hd, hd), lambda i: (0, 0)),
            pl.BlockSpec((1, hd), lambda i: (0, 0)),
            pl.BlockSpec((bn, 1), lambda i: (i, 0)),
            pl.BlockSpec((1, 1), lambda i: (0, 0)),
        ],
        out_specs=pl.BlockSpec((bn, hd), lambda i: (i, 0)),
        out_shape=jax.ShapeDtypeStruct((n, hd), jnp.float32),
    )(h, w1, b1.reshape(1, -1), w2, b2.reshape(1, -1), node_mask,
      jnp.asarray(nn, jnp.int32).reshape(1, 1))


def kernel(h0, x, edge_attr, node_mask, edge_mask, emb_w, emb_b, edge_w1,
           edge_b1, edge_w2, edge_b2, node_w1, node_b1, node_w2, node_b2,
           dec_w1, dec_b1, dec_w2, dec_b2, edges, n_nodes):
    n, _ = h0.shape
    e = edges.shape[1]
    hd = emb_w.shape[1]
    lnum = edge_w1.shape[0]

    step = CHUNK * NW
    e_pad = ((e + step - 1) // step) * step
    pad = e_pad - e
    row_p = jnp.concatenate([edges[0], jnp.zeros((pad,), edges.dtype)])
    col_p = jnp.concatenate([edges[1], jnp.zeros((pad,), edges.dtype)])
    em_p = jnp.concatenate(
        [edge_mask, jnp.zeros((pad, 1), edge_mask.dtype)], axis=0)
    ea_p = jnp.concatenate(
        [edge_attr, jnp.zeros((pad, edge_attr.shape[1]), edge_attr.dtype)], axis=0)

    radial = _sc_radial(x.reshape(-1), row_p, col_p)
    ea_aug = jnp.concatenate([radial[:, None], ea_p], axis=1)
    row_2d = row_p.reshape(-1, CHUNK)
    col_2d = col_p.reshape(-1, CHUNK)

    h = _emb(h0, emb_w, emb_b)
    for i in range(lnum):
        wa = edge_w1[i, :hd]
        wb = edge_w1[i, hd:2 * hd]
        waug = edge_w1[i, 2 * hd:]
        pa, pb = _pab(h, wa, wb)
        g = _sc_gather(pa, pb, row_2d, col_2d)
        m2 = _edge_mlp(g, ea_aug, waug, edge_b1[i], edge_w2[i],
                       edge_b2[i], em_p)
        agg = _sc_scatter(m2, row_2d, n)
        h = _node_mlp(h, agg, h0, node_w1[i, :hd], node_w1[i, hd:2 * hd],
                      node_w1[i, 2 * hd:], node_b1[i], node_w2[i], node_b2[i])

    return _dec(h, dec_w1, dec_b1, dec_w2, dec_b2, node_mask, n_nodes)
